# Initial kernel scaffold; baseline (speedup 1.0000x reference)
import jax
import jax.numpy as jnp
from jax.experimental import pallas as pl


def kernel(xyz_canon):
    raise NotImplementedError("write your pallas kernel here")


# speedup vs baseline: 16.9760x; 16.9760x over previous
import functools

import jax
import jax.numpy as jnp
from jax import lax
from jax.experimental import pallas as pl
from jax.experimental.pallas import tpu as pltpu

_N = 10000
_K = 9
_NP = 10240
_BR = 128


def _fused_body(n, k, npad, br, vfull_ref, vrow_ref, out_ref, s1, s2):
    nb = npad // br
    step = pl.program_id(0)

    @pl.when(step == 0)
    def _init():
        s2[...] = jnp.zeros_like(s2)

    @pl.when(step < nb)
    def _main():
        vp = vfull_ref[...]
        vr = vrow_ref[...]
        sq_all = jnp.sum(vp * vp, axis=1)
        sq_r = jnp.sum(vr * vr, axis=1)
        dot = lax.dot_general(
            vr, vp, (((1,), (1,)), ((), ())),
            preferred_element_type=jnp.float32,
            precision=lax.Precision.HIGHEST)
        d2 = sq_r[:, None] + sq_all[None, :] - 2.0 * dot
        cols = lax.broadcasted_iota(jnp.int32, (br, npad), 1)
        row0 = step * br
        rows_g = row0 + lax.broadcasted_iota(jnp.int32, (br, npad), 0)
        inf = jnp.float32(jnp.inf)
        d2 = jnp.where((cols == rows_g) | (cols >= n), inf, d2)

        a = jnp.zeros((br, npad), jnp.float32)
        bigi = jnp.int32(2**30)
        for _ in range(k):
            m = jnp.min(d2, axis=1)
            idx = jnp.min(jnp.where(d2 == m[:, None], cols, bigi), axis=1)
            sel = cols == idx[:, None]
            a += sel.astype(jnp.float32)
            d2 = jnp.where(sel, inf, d2)
        rvalid = (row0 + lax.broadcasted_iota(jnp.int32, (br, 1), 0)) < n
        a = jnp.where(rvalid, a, 0.0)

        one3 = (lax.broadcasted_iota(jnp.int32, (npad, 4), 1) == 3)
        v4 = vp + one3.astype(jnp.float32)
        one3r = (lax.broadcasted_iota(jnp.int32, (br, 4), 1) == 3)
        v4r = vr + one3r.astype(jnp.float32)
        s1[pl.ds(row0, br), :] = lax.dot_general(
            a, v4, (((1,), (0,)), ((), ())),
            preferred_element_type=jnp.float32,
            precision=lax.Precision.HIGHEST)
        s2[...] += lax.dot_general(
            a, v4r, (((0,), (0,)), ((), ())),
            preferred_element_type=jnp.float32,
            precision=lax.Precision.HIGHEST)

    @pl.when(step == nb)
    def _fin():
        t = s1[...] + s2[...]
        vp = vfull_ref[...]
        deg = t[:, 3:4]
        lv = t[:, 0:3] / deg - vp[:, 0:3]
        nrm = jnp.sqrt(jnp.sum(lv * lv, axis=1, keepdims=True))
        valid = lax.broadcasted_iota(jnp.int32, (npad, 1), 0) < n
        loss = jnp.sum(jnp.where(valid, nrm, 0.0)) / n
        out_ref[...] = loss[None, None]


def _fused_call(n, k, npad, br, vp, interpret=False):
    nb = npad // br
    return pl.pallas_call(
        functools.partial(_fused_body, n, k, npad, br),
        grid=(nb + 1,),
        in_specs=[
            pl.BlockSpec((npad, 4), lambda i: (0, 0)),
            pl.BlockSpec((br, 4), lambda i: (jnp.minimum(i, nb - 1), 0)),
        ],
        out_specs=pl.BlockSpec((1, 1), lambda i: (0, 0)),
        out_shape=jax.ShapeDtypeStruct((1, 1), jnp.float32),
        scratch_shapes=[
            pltpu.VMEM((npad, 4), jnp.float32),
            pltpu.VMEM((npad, 4), jnp.float32),
        ],
        interpret=interpret,
    )(vp, vp)


def kernel(xyz_canon):
    vp = jnp.zeros((_NP, 4), jnp.float32).at[:_N, :3].set(xyz_canon)
    loss = _fused_call(_N, _K, _NP, _BR, vp)
    return loss[0, 0]
